# baseline (speedup 1.0000x reference)
import jax
import jax.numpy as jnp
from jax.experimental import pallas as pl
from jax.experimental.pallas import tpu as pltpu
from jax.experimental.pallas import tpu_sc as plsc

_B, _C, _H, _W = 4, 16, 32, 32
_K = 3
_NC = 32
_D = _C * _K * _K
_MD = 2 * _D
_N = _B * _H * _W
_ITERS = 26
_SC_ROWS = 512
_TC_ROWS = _N - _SC_ROWS
_LANES = 16


def _unfold_cols(x):
    pb = (_K - 1) // 2
    pe = (_K - 1) - pb
    xp = jnp.pad(x, ((0, 0), (0, 0), (pb, pe), (pb, pe)))
    hout = xp.shape[2] - _K + 1
    wout = xp.shape[3] - _K + 1
    patches = jnp.stack([xp[:, :, i:i + hout, j:j + wout]
                         for i in range(_K) for j in range(_K)], axis=2)
    u = patches.reshape(x.shape[0], _D, hout * wout)
    return jnp.transpose(u, (1, 0, 2)).reshape(_D, -1)


def _wos_block(inp_ref, mask_ref, w_ref, bias_ref, out_ref):
    a = inp_ref[...] + mask_ref[...].reshape(_D, 1)
    v = jnp.concatenate([a, -a], axis=0)
    w = w_ref[...].reshape(_MD, 1)
    bias = bias_ref[...].reshape(1, 1)
    m = jnp.max(v, axis=0, keepdims=True)
    delta = jnp.float32(1e-3)
    lo0 = -m - delta
    hi0 = m + delta

    def body(_, carry):
        lo, hi = carry
        mid = 0.5 * (lo + hi)
        f = jnp.sum(jnp.where(v >= mid, w, 0.0), axis=0, keepdims=True)
        pred = f <= bias
        return jnp.where(pred, lo, mid), jnp.where(pred, mid, hi)

    lo, hi = jax.lax.fori_loop(0, _ITERS, body, (lo0, hi0))
    big = jnp.float32(3.0e38)
    r = jnp.min(jnp.where(v >= hi, v, big), axis=0, keepdims=True)
    r = jnp.where(r > jnp.float32(1e38), m, r)
    out_ref[...] = r.reshape(1, 1, _TC_ROWS)


def _wos_tc(inp_t, mask, weight, bias):
    mask_r = mask.reshape(_NC, _D, 1)
    weight_r = weight.reshape(_NC, _MD, 1)
    bias_r = bias.reshape(_NC, 1, 1)
    out = pl.pallas_call(
        _wos_block,
        grid=(_NC,),
        in_specs=[
            pl.BlockSpec((_D, _TC_ROWS), lambda nc: (0, 0)),
            pl.BlockSpec((1, _D, 1), lambda nc: (nc, 0, 0)),
            pl.BlockSpec((1, _MD, 1), lambda nc: (nc, 0, 0)),
            pl.BlockSpec((1, 1, 1), lambda nc: (nc, 0, 0)),
        ],
        out_specs=pl.BlockSpec((1, 1, _TC_ROWS), lambda nc: (nc, 0, 0)),
        out_shape=jax.ShapeDtypeStruct((_NC, 1, _TC_ROWS), jnp.float32),
        compiler_params=pltpu.CompilerParams(
            dimension_semantics=("arbitrary",),
        ),
    )(inp_t, mask_r, weight_r, bias_r)
    return out.reshape(_NC, _TC_ROWS)


def _wos_sc_body(inp_hbm, mask_hbm, w_hbm, bias_hbm, out_hbm,
                 inp_l, mask_l, w_l, a_l, bias_l, r_l):
    core = jax.lax.axis_index("c")
    sub = jax.lax.axis_index("s")
    unit = core * 16 + sub
    pltpu.sync_copy(inp_hbm.at[unit], inp_l)
    pltpu.sync_copy(bias_hbm, bias_l)
    zero = jnp.zeros((_LANES,), jnp.float32)
    big = jnp.full((_LANES,), 3.0e38, jnp.float32)

    @pl.loop(0, _NC)
    def _nc_loop(nc):
        pltpu.sync_copy(mask_hbm.at[nc], mask_l)
        pltpu.sync_copy(w_hbm.at[nc], w_l)

        @pl.loop(0, _D)
        def _build(d):
            a_l[d] = inp_l[d] + mask_l[d]

        def mbody(d, mcur):
            return jnp.maximum(mcur, jnp.abs(a_l[d]))
        m = jax.lax.fori_loop(0, _D, mbody, zero, unroll=8)
        delta = jnp.float32(1e-3)
        lo0 = -m - delta
        hi0 = m + delta
        bias_v = bias_l[nc]

        def bis(_, carry):
            lo, hi = carry
            mid = 0.5 * (lo + hi)
            nmid = -mid

            def fbody(d, acc):
                ad = a_l[d]
                acc = acc + jnp.where(ad >= mid, w_l[d], zero)
                return acc + jnp.where(ad <= nmid, w_l[_D + d], zero)
            f = jax.lax.fori_loop(0, _D, fbody, zero, unroll=16)
            pred = f <= bias_v
            return jnp.where(pred, lo, mid), jnp.where(pred, mid, hi)

        lo, hi = jax.lax.fori_loop(0, _ITERS, bis, (lo0, hi0))

        def rbody(d, rcur):
            ad = a_l[d]
            rcur = jnp.minimum(rcur, jnp.where(ad >= hi, ad, big))
            return jnp.minimum(rcur, jnp.where(-ad >= hi, -ad, big))
        r = jax.lax.fori_loop(0, _D, rbody, big, unroll=8)
        r = jnp.where(r > jnp.float32(1e38), m, r)
        r_l[nc] = r

    pltpu.sync_copy(r_l, out_hbm.at[unit])


def _wos_sc(inp_sc, mask, weight, bias):
    units = _SC_ROWS // _LANES
    inp_u = inp_sc.reshape(_D, units, _LANES).transpose(1, 0, 2)
    mask_rep = jnp.broadcast_to(mask[:, :, None], (_NC, _D, _LANES))
    w_rep = jnp.broadcast_to(weight[:, :, None], (_NC, _MD, _LANES))
    bias_rep = jnp.broadcast_to(bias.reshape(_NC, 1), (_NC, _LANES))
    mesh = plsc.VectorSubcoreMesh(core_axis_name="c", subcore_axis_name="s")
    run = pl.kernel(
        _wos_sc_body,
        out_type=jax.ShapeDtypeStruct((units, _NC, _LANES), jnp.float32),
        mesh=mesh,
        scratch_types=[
            pltpu.VMEM((_D, _LANES), jnp.float32),
            pltpu.VMEM((_D, _LANES), jnp.float32),
            pltpu.VMEM((_MD, _LANES), jnp.float32),
            pltpu.VMEM((_D, _LANES), jnp.float32),
            pltpu.VMEM((_NC, _LANES), jnp.float32),
            pltpu.VMEM((_NC, _LANES), jnp.float32),
        ],
    )
    out_u = run(inp_u, mask_rep, w_rep, bias_rep)
    return out_u.transpose(1, 0, 2).reshape(_NC, _SC_ROWS)


def kernel(x, mask, weight, bias):
    inp_t = _unfold_cols(x)
    y_tc = _wos_tc(inp_t[:, :_TC_ROWS], mask, weight, bias)
    y_sc = _wos_sc(inp_t[:, _TC_ROWS:], mask, weight, bias)
    y = jnp.concatenate([y_tc, y_sc], axis=1)
    return y.T.reshape(-1, _NC, _H, _W)


# speedup vs baseline: 1.0409x; 1.0409x over previous
import jax
import jax.numpy as jnp
from jax.experimental import pallas as pl
from jax.experimental.pallas import tpu as pltpu
from jax.experimental.pallas import tpu_sc as plsc

_B, _C, _H, _W = 4, 16, 32, 32
_K = 3
_NC = 32
_D = _C * _K * _K
_MD = 2 * _D
_N = _B * _H * _W
_ITERS = 26
_SC_ROWS = 512
_TC_ROWS = _N - _SC_ROWS
_LANES = 16


def _unfold_cols(x):
    pb = (_K - 1) // 2
    pe = (_K - 1) - pb
    xp = jnp.pad(x, ((0, 0), (0, 0), (pb, pe), (pb, pe)))
    hout = xp.shape[2] - _K + 1
    wout = xp.shape[3] - _K + 1
    patches = jnp.stack([xp[:, :, i:i + hout, j:j + wout]
                         for i in range(_K) for j in range(_K)], axis=2)
    u = patches.reshape(x.shape[0], _D, hout * wout)
    return jnp.transpose(u, (1, 0, 2)).reshape(_D, -1)


def _wos_block(inp_ref, mask_ref, w_ref, bias_ref, out_ref):
    inp = inp_ref[...]
    delta = jnp.float32(1e-3)
    big = jnp.float32(3.0e38)
    vs, ws, biases, ms = [], [], [], []
    for j in range(2):
        a = inp + mask_ref[...][j].reshape(_D, 1)
        vs.append(jnp.concatenate([a, -a], axis=0))
        ws.append(w_ref[...][j].reshape(_MD, 1))
        biases.append(bias_ref[...][j].reshape(1, 1))
        ms.append(jnp.max(vs[j], axis=0, keepdims=True))

    def body(_, carry):
        new = []
        for j, (lo, hi) in enumerate(zip(carry[::2], carry[1::2])):
            mid = 0.5 * (lo + hi)
            f = jnp.sum(jnp.where(vs[j] >= mid, ws[j], 0.0),
                        axis=0, keepdims=True)
            pred = f <= biases[j]
            new += [jnp.where(pred, lo, mid), jnp.where(pred, mid, hi)]
        return tuple(new)

    init = []
    for j in range(2):
        init += [-ms[j] - delta, ms[j] + delta]
    fin = jax.lax.fori_loop(0, _ITERS, body, tuple(init))
    rs = []
    for j in range(2):
        hi = fin[2 * j + 1]
        r = jnp.min(jnp.where(vs[j] >= hi, vs[j], big), axis=0, keepdims=True)
        rs.append(jnp.where(r > jnp.float32(1e38), ms[j], r))
    out_ref[...] = jnp.concatenate(rs, axis=0).reshape(2, 1, _TC_ROWS)


def _wos_tc(inp_t, mask, weight, bias):
    mask_r = mask.reshape(_NC, _D, 1)
    weight_r = weight.reshape(_NC, _MD, 1)
    bias_r = bias.reshape(_NC, 1, 1)
    out = pl.pallas_call(
        _wos_block,
        grid=(_NC // 2,),
        in_specs=[
            pl.BlockSpec((_D, _TC_ROWS), lambda nc: (0, 0)),
            pl.BlockSpec((2, _D, 1), lambda nc: (nc, 0, 0)),
            pl.BlockSpec((2, _MD, 1), lambda nc: (nc, 0, 0)),
            pl.BlockSpec((2, 1, 1), lambda nc: (nc, 0, 0)),
        ],
        out_specs=pl.BlockSpec((2, 1, _TC_ROWS), lambda nc: (nc, 0, 0)),
        out_shape=jax.ShapeDtypeStruct((_NC, 1, _TC_ROWS), jnp.float32),
        compiler_params=pltpu.CompilerParams(
            dimension_semantics=("arbitrary",),
        ),
    )(inp_t, mask_r, weight_r, bias_r)
    return out.reshape(_NC, _TC_ROWS)


def _wos_sc_body(inp_hbm, mask_hbm, w_hbm, bias_hbm, out_hbm,
                 inp_l, mask_l, w_l, a_l, bias_l, r_l):
    core = jax.lax.axis_index("c")
    sub = jax.lax.axis_index("s")
    unit = core * 16 + sub
    pltpu.sync_copy(inp_hbm.at[unit], inp_l)
    pltpu.sync_copy(bias_hbm, bias_l)
    zero = jnp.zeros((_LANES,), jnp.float32)
    big = jnp.full((_LANES,), 3.0e38, jnp.float32)

    @pl.loop(0, _NC)
    def _nc_loop(nc):
        pltpu.sync_copy(mask_hbm.at[nc], mask_l)
        pltpu.sync_copy(w_hbm.at[nc], w_l)

        @pl.loop(0, _D)
        def _build(d):
            a_l[d] = inp_l[d] + mask_l[d]

        def mbody(d, mcur):
            return jnp.maximum(mcur, jnp.abs(a_l[d]))
        m = jax.lax.fori_loop(0, _D, mbody, zero, unroll=8)
        delta = jnp.float32(1e-3)
        lo0 = -m - delta
        hi0 = m + delta
        bias_v = bias_l[nc]

        def bis(_, carry):
            lo, hi = carry
            mid = 0.5 * (lo + hi)
            nmid = -mid

            def fbody(d, acc):
                ad = a_l[d]
                acc = acc + jnp.where(ad >= mid, w_l[d], zero)
                return acc + jnp.where(ad <= nmid, w_l[_D + d], zero)
            f = jax.lax.fori_loop(0, _D, fbody, zero, unroll=16)
            pred = f <= bias_v
            return jnp.where(pred, lo, mid), jnp.where(pred, mid, hi)

        lo, hi = jax.lax.fori_loop(0, _ITERS, bis, (lo0, hi0))

        def rbody(d, rcur):
            ad = a_l[d]
            rcur = jnp.minimum(rcur, jnp.where(ad >= hi, ad, big))
            return jnp.minimum(rcur, jnp.where(-ad >= hi, -ad, big))
        r = jax.lax.fori_loop(0, _D, rbody, big, unroll=8)
        r = jnp.where(r > jnp.float32(1e38), m, r)
        r_l[nc] = r

    pltpu.sync_copy(r_l, out_hbm.at[unit])


def _wos_sc(inp_sc, mask, weight, bias):
    units = _SC_ROWS // _LANES
    inp_u = inp_sc.reshape(_D, units, _LANES).transpose(1, 0, 2)
    mask_rep = jnp.broadcast_to(mask[:, :, None], (_NC, _D, _LANES))
    w_rep = jnp.broadcast_to(weight[:, :, None], (_NC, _MD, _LANES))
    bias_rep = jnp.broadcast_to(bias.reshape(_NC, 1), (_NC, _LANES))
    mesh = plsc.VectorSubcoreMesh(core_axis_name="c", subcore_axis_name="s")
    run = pl.kernel(
        _wos_sc_body,
        out_type=jax.ShapeDtypeStruct((units, _NC, _LANES), jnp.float32),
        mesh=mesh,
        scratch_types=[
            pltpu.VMEM((_D, _LANES), jnp.float32),
            pltpu.VMEM((_D, _LANES), jnp.float32),
            pltpu.VMEM((_MD, _LANES), jnp.float32),
            pltpu.VMEM((_D, _LANES), jnp.float32),
            pltpu.VMEM((_NC, _LANES), jnp.float32),
            pltpu.VMEM((_NC, _LANES), jnp.float32),
        ],
    )
    out_u = run(inp_u, mask_rep, w_rep, bias_rep)
    return out_u.transpose(1, 0, 2).reshape(_NC, _SC_ROWS)


def kernel(x, mask, weight, bias):
    inp_t = _unfold_cols(x)
    y_tc = _wos_tc(inp_t[:, :_TC_ROWS], mask, weight, bias)
    y_sc = _wos_sc(inp_t[:, _TC_ROWS:], mask, weight, bias)
    y = jnp.concatenate([y_tc, y_sc], axis=1)
    return y.T.reshape(-1, _NC, _H, _W)
